# baseline (speedup 1.0000x reference)
import jax
import jax.numpy as jnp
from jax import lax
from jax.experimental import pallas as pl
from jax.experimental.pallas import tpu as pltpu
from jax.experimental.pallas import tpu_sc as plsc

_NC = 2
_NS = 16
_L = 16


def _sc_gather_kernel(D, x_ref, pos_ref, out_ref, idx_v, row_v, sem):
    wid = lax.axis_index("s") * _NC + lax.axis_index("c")
    q = idx_v.shape[0]
    pltpu.sync_copy(pos_ref, idx_v)
    off = wid * D
    for i in range(q // _L):
        sl = pl.ds(i * _L, _L)
        idx_v[sl] = idx_v[sl] + off
    pltpu.async_copy(x_ref.at[idx_v], row_v, sem).wait()
    pltpu.sync_copy(row_v, out_ref.at[wid])


def _gram_kernel(x_ref, m_ref, s_ref):
    g = pl.program_id(0)

    @pl.when(g == 0)
    def _init():
        s_ref[...] = jnp.zeros_like(s_ref)

    x = x_ref[...]
    bm = (m_ref[...] > 0).astype(jnp.float32)
    xb = x * bm
    x2b = x * xb
    lhs = jnp.concatenate([bm, xb, x2b], axis=0)
    rhs = jnp.concatenate([bm, xb], axis=0)
    s_ref[...] += jax.lax.dot_general(
        lhs, rhs, (((1,), (1,)), ((), ())), preferred_element_type=jnp.float32)


def _finish_kernel(s_ref, gall_ref, a_ref, out_ref):
    Tn = gall_ref.shape[0]
    Q = gall_ref.shape[1]
    R = Q // Tn
    S = s_ref[...]
    n = S[0:Tn, 0:Tn]
    p_xb = S[Tn:2 * Tn, 0:Tn]
    p_xx = S[Tn:2 * Tn, Tn:2 * Tn]
    p_x2b = S[2 * Tn:3 * Tn, 0:Tn]

    s1 = p_xb - p_xb.T
    s2 = p_x2b - 2.0 * p_xx + p_x2b.T
    n1 = jnp.maximum(n, 1.0)
    var = (s2 - s1 * s1 / n1) / jnp.maximum(n - 1.0, 1.0)
    std = jnp.sqrt(jnp.maximum(var, 0.0))

    ii = lax.broadcasted_iota(jnp.int32, (Tn, Tn), 0)
    jj = lax.broadcasted_iota(jnp.int32, (Tn, Tn), 1)
    eye = ii == jj
    dcol = jnp.sum(jnp.where(eye, n, 0.0), axis=1, keepdims=True)
    drow = jnp.sum(jnp.where(eye, n, 0.0), axis=0, keepdims=True)
    diffcount = dcol + drow - 2.0 * n
    inf = jnp.float32(jnp.inf)
    scores = jnp.where((diffcount > 0.0) & (~eye), std, inf)

    Gall = gall_ref[...]
    qi = lax.broadcasted_iota(jnp.int32, (Tn, Q), 0)
    qj = lax.broadcasted_iota(jnp.int32, (Tn, Q), 1)
    qmask = (qj // R) == qi
    a_row = a_ref[...]

    cur = scores
    negs = []
    rowsums = []
    for _ in range(3):
        mval = jnp.min(cur, axis=1, keepdims=True)
        is_min = cur == mval
        idx = jnp.min(jnp.where(is_min, jj, Tn), axis=1, keepdims=True)
        sel = (jj == idx).astype(jnp.float32)
        vk = jax.lax.dot_general(
            sel, Gall, (((1,), (0,)), ((), ())),
            preferred_element_type=jnp.float32)
        term = jnp.where(qmask, jnp.abs(a_row - vk), 0.0)
        rowsums.append(jnp.sum(term, axis=1, keepdims=True))
        negs.append(-mval)
        cur = jnp.where(jj == idx, inf, cur)

    negcat = jnp.concatenate(negs, axis=1)
    mx = jnp.max(negcat, axis=1, keepdims=True)
    e = jnp.exp(negcat - mx)
    w = e / jnp.sum(e, axis=1, keepdims=True)
    rs = jnp.concatenate(rowsums, axis=1)
    per_row = jnp.sum(w * rs, axis=1, keepdims=True)
    out_ref[...] = jnp.sum(per_row, axis=0, keepdims=True)


def kernel(X, A, M, T, nM, row_elements_pos, max_time):
    Tn, D = X.shape
    R = row_elements_pos.shape[1]
    Q = Tn * R
    CHUNK = 4096
    grid = D // CHUNK

    pos_flat = row_elements_pos.astype(jnp.int32).reshape(Q)
    a_row = A.astype(jnp.float32).reshape(1, Q)
    x_flat = X.reshape(Tn * D)

    mesh = plsc.VectorSubcoreMesh(core_axis_name="c", subcore_axis_name="s")
    gall = pl.kernel(
        lambda *refs: _sc_gather_kernel(D, *refs),
        out_type=jax.ShapeDtypeStruct((Tn, Q), jnp.float32),
        mesh=mesh,
        scratch_types=[
            pltpu.VMEM((Q,), jnp.int32),
            pltpu.VMEM((Q,), jnp.float32),
            pltpu.SemaphoreType.DMA,
        ],
    )(x_flat, pos_flat)

    gram = pl.pallas_call(
        _gram_kernel,
        grid=(grid,),
        in_specs=[
            pl.BlockSpec((Tn, CHUNK), lambda g: (0, g)),
            pl.BlockSpec((Tn, CHUNK), lambda g: (0, g)),
        ],
        out_specs=pl.BlockSpec((3 * Tn, 2 * Tn), lambda g: (0, 0)),
        out_shape=jax.ShapeDtypeStruct((3 * Tn, 2 * Tn), jnp.float32),
    )(X, M)

    out = pl.pallas_call(
        _finish_kernel,
        out_shape=jax.ShapeDtypeStruct((1, 1), jnp.float32),
    )(gram, gall, a_row)
    return jnp.reshape(out, ())


# speedup vs baseline: 2.5855x; 2.5855x over previous
import jax
import jax.numpy as jnp
from jax.experimental import pallas as pl
from jax.experimental.pallas import tpu as pltpu


def _loss_kernel(x_ref, m_ref, pos_ref, a_ref, out_ref, accg_ref, accG_ref):
    g = pl.program_id(0)
    C = x_ref.shape[1]
    Tn = x_ref.shape[0]
    Q = accG_ref.shape[1]
    R = Q // Tn

    @pl.when(g == 0)
    def _init():
        accg_ref[...] = jnp.zeros_like(accg_ref)
        accG_ref[...] = jnp.zeros_like(accG_ref)

    x = x_ref[...]
    bm = (m_ref[...] > 0).astype(jnp.float32)
    xb = x * bm
    x2b = x * xb
    lhs = jnp.concatenate([bm, xb, x2b], axis=0)
    rhs = jnp.concatenate([bm, xb], axis=0)
    accg_ref[...] += jax.lax.dot_general(
        lhs, rhs, (((1,), (1,)), ((), ())), preferred_element_type=jnp.float32)

    cols = jax.lax.broadcasted_iota(jnp.int32, (Q, C), 1) + g * C
    oh = (cols == pos_ref[...]).astype(jnp.float32)
    accG_ref[...] += jax.lax.dot_general(
        x, oh, (((1,), (1,)), ((), ())), preferred_element_type=jnp.float32)

    @pl.when(g == pl.num_programs(0) - 1)
    def _finish():
        S = accg_ref[...]
        n = S[0:Tn, 0:Tn]
        p_xb = S[Tn:2 * Tn, 0:Tn]
        p_xx = S[Tn:2 * Tn, Tn:2 * Tn]
        p_x2b = S[2 * Tn:3 * Tn, 0:Tn]

        s1 = p_xb - p_xb.T
        s2 = p_x2b - 2.0 * p_xx + p_x2b.T
        n1 = jnp.maximum(n, 1.0)
        var = (s2 - s1 * s1 / n1) / jnp.maximum(n - 1.0, 1.0)
        std = jnp.sqrt(jnp.maximum(var, 0.0))

        ii = jax.lax.broadcasted_iota(jnp.int32, (Tn, Tn), 0)
        jj = jax.lax.broadcasted_iota(jnp.int32, (Tn, Tn), 1)
        eye = ii == jj
        dcol = jnp.sum(jnp.where(eye, n, 0.0), axis=1, keepdims=True)
        drow = jnp.sum(jnp.where(eye, n, 0.0), axis=0, keepdims=True)
        diffcount = dcol + drow - 2.0 * n
        inf = jnp.float32(jnp.inf)
        scores = jnp.where((diffcount > 0.0) & (~eye), std, inf)

        Gall = accG_ref[...]
        qi = jax.lax.broadcasted_iota(jnp.int32, (Tn, Q), 0)
        qj = jax.lax.broadcasted_iota(jnp.int32, (Tn, Q), 1)
        qmask = (qj // R) == qi
        a_row = a_ref[...]

        cur = scores
        negs = []
        rowsums = []
        for _ in range(3):
            mval = jnp.min(cur, axis=1, keepdims=True)
            is_min = cur == mval
            idx = jnp.min(jnp.where(is_min, jj, Tn), axis=1, keepdims=True)
            sel = (jj == idx).astype(jnp.float32)
            vk = jax.lax.dot_general(
                sel, Gall, (((1,), (0,)), ((), ())),
                preferred_element_type=jnp.float32)
            term = jnp.where(qmask, jnp.abs(a_row - vk), 0.0)
            rowsums.append(jnp.sum(term, axis=1, keepdims=True))
            negs.append(-mval)
            cur = jnp.where(jj == idx, inf, cur)

        negcat = jnp.concatenate(negs, axis=1)
        mx = jnp.max(negcat, axis=1, keepdims=True)
        e = jnp.exp(negcat - mx)
        w = e / jnp.sum(e, axis=1, keepdims=True)
        rs = jnp.concatenate(rowsums, axis=1)
        per_row = jnp.sum(w * rs, axis=1, keepdims=True)
        out_ref[...] = jnp.sum(per_row, axis=0, keepdims=True)


def kernel(X, A, M, T, nM, row_elements_pos, max_time):
    Tn, D = X.shape
    R = row_elements_pos.shape[1]
    Q = Tn * R
    CHUNK = 8192
    grid = D // CHUNK

    pos = row_elements_pos.astype(jnp.int32).reshape(Q, 1)
    a_row = A.astype(jnp.float32).reshape(1, Q)

    out = pl.pallas_call(
        _loss_kernel,
        grid=(grid,),
        in_specs=[
            pl.BlockSpec((Tn, CHUNK), lambda g: (0, g)),
            pl.BlockSpec((Tn, CHUNK), lambda g: (0, g)),
            pl.BlockSpec((Q, 1), lambda g: (0, 0)),
            pl.BlockSpec((1, Q), lambda g: (0, 0)),
        ],
        out_specs=pl.BlockSpec((1, 1), lambda g: (0, 0)),
        out_shape=jax.ShapeDtypeStruct((1, 1), jnp.float32),
        scratch_shapes=[
            pltpu.VMEM((3 * Tn, 2 * Tn), jnp.float32),
            pltpu.VMEM((Tn, Q), jnp.float32),
        ],
    )(X, M, pos, a_row)
    return jnp.reshape(out, ())
